# baseline (speedup 1.0000x reference)
import dataclasses

import jax
import jax.numpy as jnp
from jax.experimental import pallas as pl
from jax.experimental.pallas import tpu as pltpu
from jax.experimental.pallas import tpu_sc as plsc

H = 18
W = 18
P = H * W
CIN = 45
CMID = 64
E = 512
N = 1024
NSEG = 4
BLK = 32
NB = BLK * P

_OFFS = [(k // 3 - 1, k % 3 - 1) for k in range(9)]


def _conv_block_kernel(x_ref, w1_ref, b1_ref, w2_ref, b2_ref, o_ref):
    x = x_ref[...]
    q = jax.lax.broadcasted_iota(jnp.int32, (1, NB), 1)
    ii = (q % P) // W
    jj = q % W

    def conv(inp, w_ref, b_ref, cin):
        cols = []
        for (oi, oj) in _OFFS:
            s = oi * W + oj
            xs = jnp.roll(inp, -s, axis=1) if s != 0 else inp
            valid = ((ii + oi >= 0) & (ii + oi < H)
                     & (jj + oj >= 0) & (jj + oj < W))
            cols.append(jnp.where(valid, xs, 0.0).astype(jnp.bfloat16))
        col = jnp.concatenate(cols, axis=0)
        acc = jax.lax.dot_general(
            w_ref[...], col, (((1,), (0,)), ((), ())),
            preferred_element_type=jnp.float32)
        return jax.nn.relu(acc + b_ref[...])

    y1 = conv(x, w1_ref, b1_ref, CIN)
    y2 = conv(y1, w2_ref, b2_ref, CMID)
    o_ref[...] = y2.astype(jnp.bfloat16)


def _run_convs(xt, w1m, b1, w2m, b2, interpret=False):
    grid = N // BLK
    return pl.pallas_call(
        _conv_block_kernel,
        grid=(grid,),
        in_specs=[
            pl.BlockSpec((CIN, NB), lambda i: (0, i)),
            pl.BlockSpec((CMID, 9 * CIN), lambda i: (0, 0)),
            pl.BlockSpec((CMID, 1), lambda i: (0, 0)),
            pl.BlockSpec((CMID, 9 * CMID), lambda i: (0, 0)),
            pl.BlockSpec((CMID, 1), lambda i: (0, 0)),
        ],
        out_specs=pl.BlockSpec((CMID, NB), lambda i: (0, i)),
        out_shape=jax.ShapeDtypeStruct((CMID, N * P), jnp.bfloat16),
        compiler_params=pltpu.CompilerParams(
            dimension_semantics=("parallel",)),
        interpret=interpret,
    )(xt, w1m, b1, w2m, b2)


def _linear_kernel(h_ref, w_ref, b_ref, o_ref):
    o = pl.program_id(1)

    @pl.when(o == 0)
    def _():
        o_ref[...] = jnp.broadcast_to(b_ref[...], o_ref.shape)

    o_ref[...] += jax.lax.dot_general(
        h_ref[0], w_ref[0], (((1,), (0,)), ((), ())),
        preferred_element_type=jnp.float32)


def _run_linear(hddc, w2r, lb, interpret=False):
    half = N // 2
    return pl.pallas_call(
        _linear_kernel,
        grid=(2, CMID),
        in_specs=[
            pl.BlockSpec((1, half, P), lambda n, o: (o, n, 0)),
            pl.BlockSpec((1, P, E), lambda n, o: (o, 0, 0)),
            pl.BlockSpec((1, E), lambda n, o: (0, 0)),
        ],
        out_specs=pl.BlockSpec((half, E), lambda n, o: (n, 0)),
        out_shape=jax.ShapeDtypeStruct((N, E), jnp.float32),
        compiler_params=pltpu.CompilerParams(
            dimension_semantics=("parallel", "arbitrary")),
        interpret=interpret,
    )(hddc, w2r, lb)


_LANES = 16
_CHUNK = 256


def _run_segmax(emb, seg):
    ngrp = E // _LANES
    emb_t = (emb.reshape(N, ngrp, _LANES).transpose(1, 0, 2)
             .reshape(ngrp, N // 8, 8 * _LANES))
    seg_m = seg.reshape(N // 128, 128)
    mesh = plsc.VectorSubcoreMesh(core_axis_name="c", subcore_axis_name="s")
    cp = pltpu.CompilerParams()
    if "needs_layout_passes" in pltpu.CompilerParams.__dataclass_fields__:
        cp = dataclasses.replace(cp, needs_layout_passes=False)

    @pl.kernel(
        out_type=jax.ShapeDtypeStruct((ngrp, NSEG, _LANES), jnp.float32),
        mesh=mesh,
        compiler_params=cp,
        scratch_types=[
            pltpu.VMEM((N // 8, 8 * _LANES), jnp.float32),
            pltpu.VMEM((N // 128, 128), jnp.int32),
            pltpu.VMEM((NSEG, _LANES), jnp.float32),
        ],
    )
    def segmax(emb_hbm, seg_hbm, out_hbm, buf, segs, acc):
        ci = jax.lax.axis_index("c")
        si = jax.lax.axis_index("s")
        g = ci * 16 + si
        pltpu.sync_copy(seg_hbm, segs)
        pltpu.sync_copy(emb_hbm.at[g], buf)

        zero = jnp.int32(0)
        cnt = [zero, zero, zero]
        for r in range(N // 128):
            for s8 in range(8):
                v = segs[r, pl.ds(s8 * _LANES, _LANES)]
                for k in range(1, NSEG):
                    cnt[k - 1] += jnp.sum(jnp.where(v < k, 1, 0))
        bounds = (zero, *cnt, jnp.int32(N))

        def row_max(n, a):
            v = buf[n // 8, pl.ds((n % 8) * _LANES, _LANES)]
            return jnp.maximum(a, v)

        for k in range(NSEG):
            m = jax.lax.fori_loop(
                bounds[k], bounds[k + 1], row_max,
                jnp.full((_LANES,), -jnp.inf, jnp.float32))
            acc[k] = m
        pltpu.sync_copy(acc, out_hbm.at[g])

    out_t = segmax(emb_t, seg_m)
    return out_t.transpose(1, 0, 2).reshape(NSEG, E)


def kernel(ins, outs, currents, segment_ids, conv1_w, conv1_b, conv2_w,
           conv2_b, lin_w, lin_b):
    g = jnp.concatenate([ins, outs, currents], axis=1)
    xt = g.reshape(N, CIN, P).transpose(1, 0, 2).reshape(CIN, N * P)
    w1m = conv1_w.transpose(0, 2, 3, 1).reshape(CMID, 9 * CIN).astype(jnp.bfloat16)
    w2m = conv2_w.transpose(0, 2, 3, 1).reshape(CMID, 9 * CMID).astype(jnp.bfloat16)
    b1 = conv1_b.reshape(CMID, 1)
    b2 = conv2_b.reshape(CMID, 1)
    w2r = lin_w.reshape(E, CMID, P).transpose(1, 2, 0).astype(jnp.bfloat16)
    lb = lin_b.reshape(1, E)
    seg = segment_ids.astype(jnp.int32)

    hdd = _run_convs(xt, w1m, b1, w2m, b2)
    hddc = hdd.reshape(CMID, N, P)
    emb = _run_linear(hddc, w2r, lb)
    return _run_segmax(emb, seg)
